# Initial kernel scaffold; baseline (speedup 1.0000x reference)
import jax
import jax.numpy as jnp
from jax.experimental import pallas as pl


def kernel(X, edge_index, W, W_res, gamma, beta):
    raise NotImplementedError("write your pallas kernel here")


# speedup vs baseline: 6.0261x; 6.0261x over previous
import functools

import jax
import jax.numpy as jnp
from jax import lax
from jax.experimental import pallas as pl
from jax.experimental.pallas import tpu as pltpu
from jax.experimental.pallas import tpu_sc as plsc

N = 10000
E = 320000
D = 128
NC, NS = 2, 16
NW = NC * NS
EPT = E // NW
CB = 128
NCHUNK = -(-EPT // CB)
EPT_P = NCHUNK * CB
NPAD = 10240
TRASH = N + 16
RPT = NPAD // NS
EPS = 1e-5

_mesh = plsc.VectorSubcoreMesh(core_axis_name="c", subcore_axis_name="s")


@functools.partial(
    pl.kernel,
    out_type=(
        jax.ShapeDtypeStruct((NC, NPAD), jnp.float32),
        jax.ShapeDtypeStruct((NC, NPAD), jnp.float32),
    ),
    mesh=_mesh,
    scratch_types=[
        pltpu.VMEM((NCHUNK, CB), jnp.int32),
        pltpu.VMEM((NCHUNK, CB), jnp.int32),
        pltpu.VMEM((CB,), jnp.float32),
        pltpu.VMEM_SHARED((NPAD,), jnp.float32),
        pltpu.VMEM_SHARED((NPAD,), jnp.float32),
    ],
)
def _deg_kernel(src_hbm, dst_hbm, zeros_hbm, odeg_hbm, ideg_hbm,
                sidx, didx, ones_v, osh, ish):
    c = lax.axis_index("c")
    s = lax.axis_index("s")
    wid = s * NC + c
    pltpu.sync_copy(zeros_hbm.at[pl.ds(s * RPT, RPT)], osh.at[pl.ds(s * RPT, RPT)])
    pltpu.sync_copy(zeros_hbm.at[pl.ds(s * RPT, RPT)], ish.at[pl.ds(s * RPT, RPT)])
    pltpu.sync_copy(src_hbm.at[wid], sidx)
    pltpu.sync_copy(dst_hbm.at[wid], didx)
    for i in range(CB // 16):
        ones_v[pl.ds(i * 16, 16)] = jnp.full((16,), 1.0, jnp.float32)
    plsc.subcore_barrier()

    def body(j, carry):
        pltpu.sync_copy(ones_v, osh.at[sidx.at[j]], add=True)
        pltpu.sync_copy(ones_v, ish.at[didx.at[j]], add=True)
        return carry

    lax.fori_loop(0, NCHUNK, body, 0)
    plsc.subcore_barrier()
    pltpu.sync_copy(osh.at[pl.ds(s * RPT, RPT)], odeg_hbm.at[c, pl.ds(s * RPT, RPT)])
    pltpu.sync_copy(ish.at[pl.ds(s * RPT, RPT)], ideg_hbm.at[c, pl.ds(s * RPT, RPT)])


@functools.partial(
    pl.kernel,
    out_type=jax.ShapeDtypeStruct((NC, NPAD, D), jnp.float32),
    mesh=_mesh,
    scratch_types=[
        pltpu.VMEM((NCHUNK, CB), jnp.int32),
        pltpu.VMEM((NCHUNK, CB), jnp.int32),
        pltpu.VMEM((CB, D), jnp.float32),
        pltpu.VMEM_SHARED((NPAD, D), jnp.float32),
        pltpu.SemaphoreType.DMA,
    ],
)
def _agg_kernel(src_hbm, dst_hbm, feat_hbm, zeros_hbm, out_hbm,
                sidx, didx, rows_v, agg_sh, sem):
    c = lax.axis_index("c")
    s = lax.axis_index("s")
    wid = s * NC + c
    pltpu.sync_copy(zeros_hbm.at[pl.ds(s * RPT, RPT)], agg_sh.at[pl.ds(s * RPT, RPT)])
    pltpu.sync_copy(src_hbm.at[wid], sidx)
    pltpu.sync_copy(dst_hbm.at[wid], didx)
    plsc.subcore_barrier()

    def body(j, carry):
        pltpu.async_copy(feat_hbm.at[sidx.at[j]], rows_v, sem).wait()
        pltpu.sync_copy(rows_v, agg_sh.at[didx.at[j]], add=True)
        return carry

    lax.fori_loop(0, NCHUNK, body, 0)
    plsc.subcore_barrier()
    pltpu.sync_copy(agg_sh.at[pl.ds(s * RPT, RPT)], out_hbm.at[c, pl.ds(s * RPT, RPT)])


def _scale_body(x_ref, od0_ref, od1_ref, id0_ref, id1_ref, feat_ref, invin_ref):
    odeg = jnp.maximum(od0_ref[...] + od1_ref[...], 1.0)
    ideg = jnp.maximum(id0_ref[...] + id1_ref[...], 1.0)
    feat_ref[...] = x_ref[...] * lax.rsqrt(odeg)
    invin_ref[...] = lax.rsqrt(ideg)


_scale_call = pl.pallas_call(
    _scale_body,
    out_shape=(
        jax.ShapeDtypeStruct((N, D), jnp.float32),
        jax.ShapeDtypeStruct((N, 1), jnp.float32),
    ),
)


def _final_body(p_ref, invin_ref, x_ref, w_ref, wres_ref, g_ref, b_ref, out_ref):
    agg = (p_ref[0, :N, :] + p_ref[1, :N, :]) * invin_ref[...]
    gX = jnp.maximum(jnp.dot(agg, w_ref[...], preferred_element_type=jnp.float32), 0.0)
    res = jnp.maximum(jnp.dot(x_ref[...], wres_ref[...], preferred_element_type=jnp.float32), 0.0)
    h = gX + res
    mean = jnp.mean(h, axis=0, keepdims=True)
    hm = h - mean
    var = jnp.mean(hm * hm, axis=0, keepdims=True)
    out_ref[...] = hm * lax.rsqrt(var + EPS) * g_ref[...] + b_ref[...]


_final_call = pl.pallas_call(
    _final_body,
    out_shape=jax.ShapeDtypeStruct((N, D), jnp.float32),
)


def kernel(X, edge_index, W, W_res, gamma, beta):
    src = edge_index[0].astype(jnp.int32).reshape(NW, EPT)
    dst = edge_index[1].astype(jnp.int32).reshape(NW, EPT)
    padc = jnp.full((NW, EPT_P - EPT), TRASH, jnp.int32)
    src_p = jnp.concatenate([src, padc], axis=1).reshape(NW, NCHUNK, CB)
    dst_p = jnp.concatenate([dst, padc], axis=1).reshape(NW, NCHUNK, CB)

    z1 = jnp.zeros((NPAD,), jnp.float32)
    odeg_p, ideg_p = _deg_kernel(src_p, dst_p, z1)

    od0 = odeg_p[0, :N].reshape(N, 1)
    od1 = odeg_p[1, :N].reshape(N, 1)
    id0 = ideg_p[0, :N].reshape(N, 1)
    id1 = ideg_p[1, :N].reshape(N, 1)
    feat, inv_in = _scale_call(X, od0, od1, id0, id1)

    feat_pad = jnp.pad(feat, ((0, NPAD - N), (0, 0)))
    z2 = jnp.zeros((NPAD, D), jnp.float32)
    p = _agg_kernel(src_p, dst_p, feat_pad, z2)

    return _final_call(p, inv_in, X, W, W_res,
                       gamma.reshape(1, D), beta.reshape(1, D))
